# Initial kernel scaffold; baseline (speedup 1.0000x reference)
import jax
import jax.numpy as jnp
from jax.experimental import pallas as pl


def kernel(x, edge_index, batch, conv_w0, conv_b0, gn_w0, gn_b0, gn_ms0, conv_w1, conv_b1, gn_w1, gn_b1, gn_ms1, conv_w2, conv_b2, gn_w2, gn_b2, gn_ms2, fc1_w, fc1_b, bn_w, bn_b, bn_rm, bn_rv, fc2_w, fc2_b):
    raise NotImplementedError("write your pallas kernel here")


# speedup vs baseline: 10.5487x; 10.5487x over previous
import functools

import jax
import jax.numpy as jnp
from jax import lax
from jax.experimental import pallas as pl
from jax.experimental.pallas import tpu as pltpu
from jax.experimental.pallas import tpu_sc as plsc

N = 10000
E = 320000
D = 128
G = 64
NCLS = 10
EPS = 1e-5

NC = 2
NS = 16
CH = 128
EPT = ((E // (NC * NS) + CH - 1) // CH) * CH
NCH = EPT // CH
EPAD = NC * NS * EPT
NP = 10112
RPT = NP // NS

def _mesh():
    return plsc.VectorSubcoreMesh(core_axis_name="c", subcore_axis_name="s",
                                  num_cores=NC, num_subcores=NS)


_HI = lax.Precision.HIGHEST


def _sc_deg(dstb, ones, zeros16):
    @functools.partial(
        pl.kernel,
        mesh=_mesh(),
        out_type=jax.ShapeDtypeStruct((NC, NP, 16), jnp.float32),
        scratch_types=[
            pltpu.VMEM((NCH, CH), jnp.int32),
            pltpu.VMEM((CH, 16), jnp.float32),
            pltpu.VMEM_SHARED((NP, 16), jnp.float32),
        ],
    )
    def k(dr, ones_hbm, z16, out, d_loc, ones_v, acc):
        c = lax.axis_index("c")
        s = lax.axis_index("s")
        pltpu.sync_copy(dr.at[c, s], d_loc)
        pltpu.sync_copy(ones_hbm, ones_v)
        rows = pl.ds(s * RPT, RPT)
        pltpu.sync_copy(z16.at[rows], acc.at[rows])
        plsc.subcore_barrier()

        @pl.loop(0, NCH)
        def _(i):
            pltpu.sync_copy(ones_v, acc.at[d_loc.at[i]], add=True)

        plsc.subcore_barrier()
        pltpu.sync_copy(acc.at[rows], out.at[c, rows])

    return k(dstb, ones, zeros16)


def _sc_scatter(table, srcb, dstb, zeros):
    @functools.partial(
        pl.kernel,
        mesh=_mesh(),
        out_type=jax.ShapeDtypeStruct((NC, NP, D), jnp.float32),
        scratch_types=[
            pltpu.VMEM((NCH, CH), jnp.int32),
            pltpu.VMEM((NCH, CH), jnp.int32),
            pltpu.VMEM((CH, D), jnp.float32),
            pltpu.VMEM_SHARED((NP, D), jnp.float32),
            pltpu.SemaphoreType.DMA,
        ],
    )
    def k(tbl, sr, dr, zr, out, s_loc, d_loc, rows, acc, sem):
        c = lax.axis_index("c")
        s = lax.axis_index("s")
        pltpu.sync_copy(sr.at[c, s], s_loc)
        pltpu.sync_copy(dr.at[c, s], d_loc)
        rslc = pl.ds(s * RPT, RPT)
        pltpu.sync_copy(zr.at[rslc], acc.at[rslc])
        plsc.subcore_barrier()

        @pl.loop(0, NCH)
        def _(i):
            pltpu.async_copy(tbl.at[s_loc.at[i]], rows, sem).wait()
            pltpu.sync_copy(rows, acc.at[d_loc.at[i]], add=True)

        plsc.subcore_barrier()
        pltpu.sync_copy(acc.at[rslc], out.at[c, rslc])

    return k(table, srcb, dstb, zeros)


def _dotT(a, b):
    return lax.dot_general(a, b, (((0,), (0,)), ((), ())), precision=_HI,
                           preferred_element_type=jnp.float32)


def _dot(a, b):
    return jnp.dot(a, b, precision=_HI, preferred_element_type=jnp.float32)


R = 2000
NR = N // R


def _indicator(batch2):
    gid = lax.broadcasted_iota(jnp.int32, (batch2.shape[0], G), 1)
    return (batch2 == gid).astype(jnp.float32)


def _tc_pre_body(x_ref, w_ref, degp_ref, hp_ref, dinv_ref):
    deg = degp_ref[0, 0:N, 0:1] + degp_ref[1, 0:N, 0:1] + 1.0
    dinv = lax.rsqrt(deg)
    hp_ref[...] = _dot(x_ref[...], w_ref[...]) * dinv
    dinv_ref[...] = dinv


def _tc_pre(x, w, degp):
    return pl.pallas_call(
        _tc_pre_body,
        out_shape=(
            jax.ShapeDtypeStruct((N, D), jnp.float32),
            jax.ShapeDtypeStruct((N, 1), jnp.float32),
        ),
    )(x, w, degp)


def _tc_stats_body(parts_ref, hp_ref, dinv_ref, batch_ref, bconv_ref,
                   out_ref, s1_ref, s2_ref, cnt_ref):
    out = dinv_ref[...] * (parts_ref[0] + parts_ref[1]
                           + hp_ref[...]) + bconv_ref[...]
    out_ref[...] = out
    ind = _indicator(batch_ref[...])
    pc = _dotT(ind, jnp.ones((R, 1), jnp.float32))
    p1 = _dotT(ind, out)
    p2 = _dotT(ind, out * out)

    @pl.when(pl.program_id(0) == 0)
    def _():
        cnt_ref[...] = pc
        s1_ref[...] = p1
        s2_ref[...] = p2

    @pl.when(pl.program_id(0) > 0)
    def _():
        cnt_ref[...] += pc
        s1_ref[...] += p1
        s2_ref[...] += p2


def _tc_stats(parts, hp, dinv, batch2, bconv):
    return pl.pallas_call(
        _tc_stats_body,
        grid=(NR,),
        in_specs=[
            pl.BlockSpec((NC, R, D), lambda i: (0, i, 0)),
            pl.BlockSpec((R, D), lambda i: (i, 0)),
            pl.BlockSpec((R, 1), lambda i: (i, 0)),
            pl.BlockSpec((R, 1), lambda i: (i, 0)),
            pl.BlockSpec((D,), lambda i: (0,)),
        ],
        out_specs=(
            pl.BlockSpec((R, D), lambda i: (i, 0)),
            pl.BlockSpec((G, D), lambda i: (0, 0)),
            pl.BlockSpec((G, D), lambda i: (0, 0)),
            pl.BlockSpec((G, 1), lambda i: (0, 0)),
        ),
        out_shape=(
            jax.ShapeDtypeStruct((N, D), jnp.float32),
            jax.ShapeDtypeStruct((G, D), jnp.float32),
            jax.ShapeDtypeStruct((G, D), jnp.float32),
            jax.ShapeDtypeStruct((G, 1), jnp.float32),
        ),
    )(parts, hp, dinv, batch2, bconv)


def _norm_rows(out, batch2, s1, s2, cnt, gnw, gnb, gnms):
    cnt = jnp.maximum(cnt, 1.0)
    mean = s1 / cnt
    var = (s2 - cnt * mean * mean * gnms * (2.0 - gnms)) / cnt
    inv_std = 1.0 / jnp.sqrt(var + EPS)
    ind = _indicator(batch2)
    mean_rows = _dot(ind, mean * gnms)
    inv_rows = _dot(ind, inv_std)
    y = gnw * (out - mean_rows) * inv_rows + gnb
    return jnp.where(y > 0, y, jnp.exp(jnp.minimum(y, 0.0)) - 1.0)


def _tc_norm_mid_body(out_ref, xres_ref, dinv_ref, batch_ref, s1_ref, s2_ref,
                      cnt_ref, gnw_ref, gnb_ref, gnms_ref, wnext_ref,
                      xn_ref, hpn_ref):
    e = _norm_rows(out_ref[...], batch_ref[...], s1_ref[...], s2_ref[...],
                   cnt_ref[...], gnw_ref[...], gnb_ref[...], gnms_ref[...])
    xn = e + xres_ref[...]
    xn_ref[...] = xn
    hpn_ref[...] = _dot(xn, wnext_ref[...]) * dinv_ref[...]


def _tc_norm_mid(out, xres, dinv, batch2, s1, s2, cnt, gnw, gnb, gnms, wnext):
    return pl.pallas_call(
        _tc_norm_mid_body,
        grid=(NR,),
        in_specs=[
            pl.BlockSpec((R, D), lambda i: (i, 0)),
            pl.BlockSpec((R, D), lambda i: (i, 0)),
            pl.BlockSpec((R, 1), lambda i: (i, 0)),
            pl.BlockSpec((R, 1), lambda i: (i, 0)),
            pl.BlockSpec((G, D), lambda i: (0, 0)),
            pl.BlockSpec((G, D), lambda i: (0, 0)),
            pl.BlockSpec((G, 1), lambda i: (0, 0)),
            pl.BlockSpec((D,), lambda i: (0,)),
            pl.BlockSpec((D,), lambda i: (0,)),
            pl.BlockSpec((D,), lambda i: (0,)),
            pl.BlockSpec((D, D), lambda i: (0, 0)),
        ],
        out_specs=(
            pl.BlockSpec((R, D), lambda i: (i, 0)),
            pl.BlockSpec((R, D), lambda i: (i, 0)),
        ),
        out_shape=(
            jax.ShapeDtypeStruct((N, D), jnp.float32),
            jax.ShapeDtypeStruct((N, D), jnp.float32),
        ),
    )(out, xres, dinv, batch2, s1, s2, cnt, gnw, gnb, gnms, wnext)


def _tc_norm_last_body(out_ref, xres_ref, batch_ref, s1_ref, s2_ref, cnt_ref,
                       gnw_ref, gnb_ref, gnms_ref, xf_ref):
    e = _norm_rows(out_ref[...], batch_ref[...], s1_ref[...], s2_ref[...],
                   cnt_ref[...], gnw_ref[...], gnb_ref[...], gnms_ref[...])
    xf_ref[...] = e + xres_ref[...]


def _tc_norm_last(out, xres, batch2, s1, s2, cnt, gnw, gnb, gnms):
    return pl.pallas_call(
        _tc_norm_last_body,
        grid=(NR,),
        in_specs=[
            pl.BlockSpec((R, D), lambda i: (i, 0)),
            pl.BlockSpec((R, D), lambda i: (i, 0)),
            pl.BlockSpec((R, 1), lambda i: (i, 0)),
            pl.BlockSpec((G, D), lambda i: (0, 0)),
            pl.BlockSpec((G, D), lambda i: (0, 0)),
            pl.BlockSpec((G, 1), lambda i: (0, 0)),
            pl.BlockSpec((D,), lambda i: (0,)),
            pl.BlockSpec((D,), lambda i: (0,)),
            pl.BlockSpec((D,), lambda i: (0,)),
        ],
        out_specs=pl.BlockSpec((R, D), lambda i: (i, 0)),
        out_shape=jax.ShapeDtypeStruct((N, D), jnp.float32),
    )(out, xres, batch2, s1, s2, cnt, gnw, gnb, gnms)


def _tc_head_body(xf_ref, batch_ref, fc1w_ref, fc1b_ref, bnw_ref, bnb_ref,
                  bnrm_ref, bnrv_ref, fc2w_ref, fc2b_ref, z_ref):
    ind = _indicator(batch_ref[...])
    pooled = _dotT(ind, xf_ref[...])
    h = _dot(pooled, fc1w_ref[...]) + fc1b_ref[...]
    h = bnw_ref[...] * (h - bnrm_ref[...]) / jnp.sqrt(bnrv_ref[...] + EPS) \
        + bnb_ref[...]
    h = jnp.maximum(h, 0.0)
    z_ref[...] = _dot(h, fc2w_ref[...]) + fc2b_ref[...]


def _tc_head(xf, batch2, fc1_w, fc1_b, bn_w, bn_b, bn_rm, bn_rv, fc2_w, fc2_b):
    return pl.pallas_call(
        _tc_head_body,
        out_shape=jax.ShapeDtypeStruct((G, NCLS), jnp.float32),
    )(xf, batch2, fc1_w, fc1_b, bn_w, bn_b, bn_rm, bn_rv, fc2_w, fc2_b)


def kernel(x, edge_index, batch, conv_w0, conv_b0, gn_w0, gn_b0, gn_ms0,
           conv_w1, conv_b1, gn_w1, gn_b1, gn_ms1, conv_w2, conv_b2, gn_w2,
           gn_b2, gn_ms2, fc1_w, fc1_b, bn_w, bn_b, bn_rm, bn_rv, fc2_w,
           fc2_b):
    src = edge_index[0].astype(jnp.int32)
    dst = edge_index[1].astype(jnp.int32)
    pad = EPAD - E
    srcb = jnp.concatenate([src, jnp.zeros((pad,), jnp.int32)])
    dstb = jnp.concatenate([dst, jnp.full((pad,), N, jnp.int32)])
    srcb = srcb.reshape(NC, NS, NCH, CH)
    dstb = dstb.reshape(NC, NS, NCH, CH)
    batch2 = batch.astype(jnp.int32).reshape(N, 1)

    zeros128 = jnp.zeros((NP, D), jnp.float32)
    zeros16 = jnp.zeros((NP, 16), jnp.float32)
    ones16 = jnp.ones((CH, 16), jnp.float32)

    degp = _sc_deg(dstb, ones16, zeros16)
    hp, dinv = _tc_pre(x, conv_w0, degp)

    p = _sc_scatter(hp, srcb, dstb, zeros128)
    out0, s1, s2, cnt = _tc_stats(p, hp, dinv, batch2, conv_b0)
    x1, hp1 = _tc_norm_mid(out0, x, dinv, batch2, s1, s2, cnt, gn_w0, gn_b0,
                           gn_ms0, conv_w1)
    p1 = _sc_scatter(hp1, srcb, dstb, zeros128)
    out1, s1, s2, cnt = _tc_stats(p1, hp1, dinv, batch2, conv_b1)
    x2, hp2 = _tc_norm_mid(out1, x1, dinv, batch2, s1, s2, cnt, gn_w1, gn_b1,
                           gn_ms1, conv_w2)
    p2 = _sc_scatter(hp2, srcb, dstb, zeros128)
    out2, s1, s2, cnt = _tc_stats(p2, hp2, dinv, batch2, conv_b2)
    xf = _tc_norm_last(out2, x2, batch2, s1, s2, cnt, gn_w2, gn_b2, gn_ms2)
    z = _tc_head(xf, batch2, fc1_w, fc1_b, bn_w, bn_b, bn_rm, bn_rv,
                 fc2_w, fc2_b)
    return (xf, z)
